# baseline (speedup 1.0000x reference)
import functools

import jax
import jax.numpy as jnp
from jax import lax
from jax.experimental import pallas as pl
from jax.experimental.pallas import tpu as pltpu
from jax.experimental.pallas import tpu_sc as plsc

NC = 2
NS = 16
NW = NC * NS
IDX_W = 128
STREAMS = 8
CHUNK = IDX_W * STREAMS


@functools.partial(jax.jit, static_argnums=(2, 3))
def _sc_gather(table, idx_flat, b, dim):
    b_per_w = b // NW
    n_chunks = b_per_w // CHUNK
    idx2d = idx_flat.reshape(b // IDX_W, IDX_W)

    mesh = plsc.VectorSubcoreMesh(core_axis_name="c", subcore_axis_name="s")

    @functools.partial(
        pl.kernel,
        mesh=mesh,
        out_type=jax.ShapeDtypeStruct((b, dim), jnp.float32),
        scratch_types=[
            pltpu.VMEM((STREAMS, IDX_W), jnp.int32),
            pltpu.VMEM((CHUNK, dim), jnp.float32),
            pltpu.SemaphoreType.DMA,
        ],
        compiler_params=pltpu.CompilerParams(use_tc_tiling_on_sc=False),
    )
    def k(table_hbm, idx_hbm, out_hbm, idx_v, rows_v, sem):
        wid = lax.axis_index("s") * NC + lax.axis_index("c")
        base = wid * b_per_w

        @pl.loop(0, n_chunks)
        def _(j):
            off = pl.multiple_of(base + j * CHUNK, CHUNK)
            pltpu.sync_copy(
                idx_hbm.at[pl.ds(pl.multiple_of(off // IDX_W, STREAMS), STREAMS)],
                idx_v,
            )
            copies = []
            for i in range(STREAMS):
                copies.append(
                    pltpu.async_copy(
                        table_hbm.at[idx_v.at[i]],
                        rows_v.at[pl.ds(i * IDX_W, IDX_W)],
                        sem,
                    )
                )
            for c in copies:
                c.wait()
            pltpu.sync_copy(rows_v, out_hbm.at[pl.ds(off, CHUNK)])

    return k(table, idx2d)


def kernel(input_ids, embedding_matrix):
    batch, seq = input_ids.shape
    dim = embedding_matrix.shape[1]
    b = batch * seq
    idx = input_ids.reshape(-1).astype(jnp.int32)
    out = _sc_gather(embedding_matrix, idx, b, dim)
    return out.reshape(batch, seq, dim)


# speedup vs baseline: 1.0078x; 1.0078x over previous
import functools

import jax
import jax.numpy as jnp
from jax import lax
from jax.experimental import pallas as pl
from jax.experimental.pallas import tpu as pltpu
from jax.experimental.pallas import tpu_sc as plsc

NC = 2
NS = 16
NW = NC * NS
IDX_W = 128
STREAMS = 4
CHUNK = IDX_W * STREAMS
NBUF = 2


@functools.partial(jax.jit, static_argnums=(2, 3))
def _sc_gather(table, idx_flat, b, dim):
    b_per_w = b // NW
    rows_per_w = b_per_w // IDX_W
    n_chunks = b_per_w // CHUNK
    idx2d = idx_flat.reshape(b // IDX_W, IDX_W)

    mesh = plsc.VectorSubcoreMesh(core_axis_name="c", subcore_axis_name="s")

    @functools.partial(
        pl.kernel,
        mesh=mesh,
        out_type=jax.ShapeDtypeStruct((b, dim), jnp.float32),
        scratch_types=[
            pltpu.VMEM((rows_per_w, IDX_W), jnp.int32),
            pltpu.VMEM((NBUF, CHUNK, dim), jnp.float32),
            pltpu.SemaphoreType.DMA((NBUF,)),
            pltpu.SemaphoreType.DMA((NBUF,)),
            pltpu.SemaphoreType.DMA,
        ],
        compiler_params=pltpu.CompilerParams(use_tc_tiling_on_sc=False),
    )
    def k(table_hbm, idx_hbm, out_hbm, idx_v, rows_v, gsem, wsem, isem):
        wid = lax.axis_index("s") * NC + lax.axis_index("c")
        base = wid * b_per_w

        pltpu.async_copy(
            idx_hbm.at[pl.ds(pl.multiple_of(base // IDX_W, 8), rows_per_w)],
            idx_v,
            isem,
        ).wait()

        def fire_gathers(c, buf):
            copies = []
            for i in range(STREAMS):
                copies.append(
                    pltpu.async_copy(
                        table_hbm.at[idx_v.at[c * STREAMS + i]],
                        rows_v.at[buf, pl.ds(i * IDX_W, IDX_W)],
                        gsem.at[buf],
                    )
                )
            return copies

        def fire_writeout(c, buf):
            off = pl.multiple_of(base + c * CHUNK, CHUNK)
            return pltpu.async_copy(
                rows_v.at[buf], out_hbm.at[pl.ds(off, CHUNK)], wsem.at[buf]
            )

        @pl.loop(0, n_chunks, step=NBUF)
        def _(j):
            gathers = [fire_gathers(j + bf, bf) for bf in range(NBUF)]
            writes = []
            for bf in range(NBUF):
                for g in gathers[bf]:
                    g.wait()
                writes.append(fire_writeout(j + bf, bf))
            for w in writes:
                w.wait()

    return k(table, idx2d)


def kernel(input_ids, embedding_matrix):
    batch, seq = input_ids.shape
    dim = embedding_matrix.shape[1]
    b = batch * seq
    idx = input_ids.reshape(-1).astype(jnp.int32)
    out = _sc_gather(embedding_matrix, idx, b, dim)
    return out.reshape(batch, seq, dim)
